# baseline (speedup 1.0000x reference)
import functools

import jax
import jax.numpy as jnp
from jax import lax
from jax.experimental import pallas as pl
from jax.experimental.pallas import tpu as pltpu
from jax.experimental.pallas import tpu_sc as plsc

N = 10000
D = 128
E = 320000

NC, NS = 2, 16
HALF = 5120
HALF_PAD = HALF + 8
TRASH = HALF
EPT = E // NS
K = 80
CMAX = EPT // K
NBUF = 2
LEAD = 1
ZR = HALF // NS
PKCAP = EPT + K
SHIFT = 13
MASK = (1 << SHIFT) - 1

_SC_PARAMS = pltpu.CompilerParams(needs_layout_passes=False)


def _make_agg(with_counts):
  mesh = plsc.VectorSubcoreMesh(core_axis_name="c", subcore_axis_name="s")

  out_type = [jax.ShapeDtypeStruct((NC, HALF, D), jnp.float32)]
  scratch = [
      pltpu.VMEM((EPT,), jnp.int32),
      pltpu.VMEM((EPT,), jnp.int32),
      pltpu.VMEM((PKCAP,), jnp.int32),
      pltpu.VMEM((NBUF, K), jnp.int32),
      pltpu.VMEM((NBUF, K), jnp.int32),
      pltpu.VMEM((NBUF, K, D), jnp.float32),
      pltpu.VMEM_SHARED((HALF_PAD, D), jnp.float32),
      pltpu.SemaphoreType.DMA,
      pltpu.SemaphoreType.DMA,
  ]
  if with_counts:
    out_type.append(jax.ShapeDtypeStruct((NC, 1, HALF), jnp.float32))
    scratch += [
        pltpu.VMEM((K,), jnp.float32),
        pltpu.VMEM_SHARED((HALF_PAD,), jnp.float32),
    ]

  def body(args):
    if with_counts:
      (table, src_h, dst_h, zeros, zeros1, out, out_cnt,
       sflat, dflat, packed, sidx, didx, rows, acc, gsem, ssem,
       ones_v, cnt1) = args
    else:
      (table, src_h, dst_h, zeros, out,
       sflat, dflat, packed, sidx, didx, rows, acc, gsem, ssem) = args
    cid = lax.axis_index("c")
    sid = lax.axis_index("s")

    pltpu.sync_copy(src_h.at[sid], sflat)
    pltpu.sync_copy(dst_h.at[sid], dflat)
    pltpu.sync_copy(zeros, acc.at[pl.ds(sid * ZR, ZR)])
    if with_counts:
      @pl.when(sid == 0)
      def _():
        pltpu.sync_copy(zeros1, cnt1.at[pl.ds(0, HALF)])
      ov = jnp.ones((16,), jnp.float32)
      for r in range(K // 16):
        ones_v[pl.ds(r * 16, 16)] = ov

    lo = cid * HALF

    def compact(i, off):
      s16 = sflat[pl.ds(i * 16, 16)]
      d16 = dflat[pl.ds(i * 16, 16)]
      vl = d16 - lo
      m = jnp.logical_and(vl >= 0, vl < HALF)
      pk = jnp.bitwise_or(lax.shift_left(s16, SHIFT),
                          jnp.bitwise_and(vl, MASK))
      plsc.store_compressed(packed.at[pl.ds(off, 16)], pk, mask=m)
      return off + jnp.sum(jnp.where(m, 1, 0))

    count = pl.loop(0, EPT // 16, init_carry=jnp.int32(0))(compact)
    dummy = jnp.full((16,), TRASH, jnp.int32)
    for j in range(K // 16):
      packed[pl.ds(count + j * 16, 16)] = dummy
    nch = (count + (K - 1)) // K

    plsc.subcore_barrier()

    def unpack(c, b):
      base = c * K
      for j in range(K // 16):
        p = packed[pl.ds(base + j * 16, 16)]
        sidx[b, pl.ds(j * 16, 16)] = lax.shift_right_logical(p, SHIFT)
        didx[b, pl.ds(j * 16, 16)] = jnp.bitwise_and(p, MASK)

    def wait_gather(b):
      pltpu.make_async_copy(table.at[sidx.at[b]], rows.at[b], gsem).wait()

    def wait_scatter(b):
      pltpu.make_async_copy(rows.at[b], acc.at[didx.at[b]], ssem).wait()

    for b in range(LEAD):
      @pl.when(b < nch)
      def _():
        unpack(b, b)
        pltpu.async_copy(table.at[sidx.at[b]], rows.at[b], gsem)

    @pl.loop(0, CMAX, step=NBUF)
    def _main(c0):
      for b in range(NBUF):
        c = c0 + b

        @pl.when(c < nch)
        def _():
          wait_gather(b)
          pltpu.async_copy(rows.at[b], acc.at[didx.at[b]], ssem, add=True)
          if with_counts:
            pltpu.sync_copy(ones_v, cnt1.at[didx.at[b]], add=True)

        bf = (b + LEAD) % NBUF

        @pl.when(c + LEAD < nch)
        def _():
          @pl.when(c >= LEAD)
          def _():
            wait_scatter(bf)
          unpack(c + LEAD, bf)
          pltpu.async_copy(table.at[sidx.at[bf]], rows.at[bf], gsem)

    for b in range(NBUF):
      @pl.when(jnp.logical_and(nch >= 1, (nch - 1) % NBUF == b))
      def _():
        wait_scatter(b)

    plsc.subcore_barrier()
    pltpu.sync_copy(acc.at[pl.ds(sid * ZR, ZR)],
                    out.at[cid, pl.ds(sid * ZR, ZR)])
    if with_counts:
      @pl.when(sid == 0)
      def _():
        pltpu.sync_copy(cnt1.at[pl.ds(0, HALF)], out_cnt.at[cid, 0])

  if with_counts:
    def agg(table, src_h, dst_h, zeros, zeros1, out, out_cnt, *scr):
      body((table, src_h, dst_h, zeros, zeros1, out, out_cnt) + scr)
  else:
    def agg(table, src_h, dst_h, zeros, out, *scr):
      body((table, src_h, dst_h, zeros, out) + scr)
    out_type = out_type[0]

  return pl.kernel(agg, out_type=out_type, mesh=mesh, scratch_types=scratch,
                   compiler_params=_SC_PARAMS)


_agg_cnt = _make_agg(True)
_agg = _make_agg(False)

R = 1000
GRID = N // R


def _dense1_body(acc, cnt, x, wl, wr, b, h_out, inv_out):
  inv = 1.0 / jnp.maximum(cnt[...], 1.0)
  agg = acc[...] * inv
  dn = (((1,), (1,)), ((), ()))
  y = (lax.dot_general(agg, wl[...], dn, preferred_element_type=jnp.float32)
       + lax.dot_general(x[...], wr[...], dn, preferred_element_type=jnp.float32)
       + b[...])
  h_out[...] = jnp.maximum(y, 0.0)
  inv_out[...] = jnp.broadcast_to(inv, (R, 8))


def _dense2_body(acc, h, inv8, wl, wr, b, out):
  agg = acc[...] * inv8[:, :1]
  dn = (((1,), (1,)), ((), ()))
  out[...] = (lax.dot_general(agg, wl[...], dn, preferred_element_type=jnp.float32)
              + lax.dot_general(h[...], wr[...], dn, preferred_element_type=jnp.float32)
              + b[...])


_dense1 = pl.pallas_call(
    _dense1_body,
    grid=(GRID,),
    in_specs=[
        pl.BlockSpec((R, D), lambda i: (i, 0)),
        pl.BlockSpec((R, 1), lambda i: (i, 0)),
        pl.BlockSpec((R, D), lambda i: (i, 0)),
        pl.BlockSpec((D, D), lambda i: (0, 0)),
        pl.BlockSpec((D, D), lambda i: (0, 0)),
        pl.BlockSpec((1, D), lambda i: (0, 0)),
    ],
    out_specs=[
        pl.BlockSpec((R, D), lambda i: (i, 0)),
        pl.BlockSpec((R, 8), lambda i: (i, 0)),
    ],
    out_shape=[
        jax.ShapeDtypeStruct((N, D), jnp.float32),
        jax.ShapeDtypeStruct((N, 8), jnp.float32),
    ],
)

_dense2 = pl.pallas_call(
    _dense2_body,
    grid=(GRID,),
    in_specs=[
        pl.BlockSpec((R, D), lambda i: (i, 0)),
        pl.BlockSpec((R, D), lambda i: (i, 0)),
        pl.BlockSpec((R, 8), lambda i: (i, 0)),
        pl.BlockSpec((D, D), lambda i: (0, 0)),
        pl.BlockSpec((D, D), lambda i: (0, 0)),
        pl.BlockSpec((1, D), lambda i: (0, 0)),
    ],
    out_specs=pl.BlockSpec((R, D), lambda i: (i, 0)),
    out_shape=jax.ShapeDtypeStruct((N, D), jnp.float32),
)


def kernel(x, edge_index, W1l, b1l, W1r, b1r, W2l, b2l, W2r, b2r):
  src_h = edge_index[0].astype(jnp.int32).reshape(NS, EPT)
  dst_h = edge_index[1].astype(jnp.int32).reshape(NS, EPT)
  zeros = jnp.zeros((ZR, D), jnp.float32)
  zeros1 = jnp.zeros((HALF,), jnp.float32)

  accp1, cntp = _agg_cnt(x, src_h, dst_h, zeros, zeros1)
  acc1 = accp1.reshape(NC * HALF, D)
  cnt = cntp.reshape(NC * HALF, 1)
  h, inv8 = _dense1(acc1, cnt, x, W1l, W1r, (b1l + b1r)[None, :])
  accp2 = _agg(h, src_h, dst_h, zeros)
  acc2 = accp2.reshape(NC * HALF, D)
  out = _dense2(acc2, h, inv8, W2l, W2r, (b2l + b2r)[None, :])
  return out


# speedup vs baseline: 1.2834x; 1.2834x over previous
import functools

import jax
import jax.numpy as jnp
from jax import lax
from jax.experimental import pallas as pl
from jax.experimental.pallas import tpu as pltpu
from jax.experimental.pallas import tpu_sc as plsc

N = 10000
D = 128
E = 320000

NC, NS = 2, 16
HALF = 5120
HALF_PAD = HALF + 8
TRASH = HALF
EPT = E // NS
K = 80
CMAX = EPT // K
NBUF = 3
LEAD = 2
SEG = 2
EPS = EPT // SEG
ZR = HALF // NS
PKCAP = EPT + K
SHIFT = 13
MASK = (1 << SHIFT) - 1

_SC_PARAMS = pltpu.CompilerParams(needs_layout_passes=False)


def _make_agg(with_counts):
  mesh = plsc.VectorSubcoreMesh(core_axis_name="c", subcore_axis_name="s")

  out_type = [jax.ShapeDtypeStruct((NC, HALF, D), jnp.float32)]
  scratch = [
      pltpu.VMEM((EPS,), jnp.int32),
      pltpu.VMEM((EPS,), jnp.int32),
      pltpu.VMEM((PKCAP,), jnp.int32),
      pltpu.VMEM((NBUF, K), jnp.int32),
      pltpu.VMEM((NBUF, K), jnp.int32),
      pltpu.VMEM((NBUF, K, D), jnp.float32),
      pltpu.VMEM_SHARED((HALF_PAD, D), jnp.float32),
      pltpu.SemaphoreType.DMA,
      pltpu.SemaphoreType.DMA,
  ]
  if with_counts:
    out_type.append(jax.ShapeDtypeStruct((NC, 1, HALF), jnp.float32))
    scratch += [
        pltpu.VMEM((K,), jnp.float32),
        pltpu.VMEM_SHARED((HALF_PAD,), jnp.float32),
    ]

  def body(args):
    if with_counts:
      (table, src_h, dst_h, zeros, zeros1, out, out_cnt,
       sflat, dflat, packed, sidx, didx, rows, acc, gsem, ssem,
       ones_v, cnt1) = args
    else:
      (table, src_h, dst_h, zeros, out,
       sflat, dflat, packed, sidx, didx, rows, acc, gsem, ssem) = args
    cid = lax.axis_index("c")
    sid = lax.axis_index("s")

    pltpu.sync_copy(zeros, acc.at[pl.ds(sid * ZR, ZR)])
    if with_counts:
      @pl.when(sid == 0)
      def _():
        pltpu.sync_copy(zeros1, cnt1.at[pl.ds(0, HALF)])
      ov = jnp.ones((16,), jnp.float32)
      for r in range(K // 16):
        ones_v[pl.ds(r * 16, 16)] = ov

    lo = cid * HALF

    def compact(i, off):
      s16 = sflat[pl.ds(i * 16, 16)]
      d16 = dflat[pl.ds(i * 16, 16)]
      vl = d16 - lo
      m = jnp.logical_and(vl >= 0, vl < HALF)
      pk = jnp.bitwise_or(lax.shift_left(s16, SHIFT),
                          jnp.bitwise_and(vl, MASK))
      plsc.store_compressed(packed.at[pl.ds(off, 16)], pk, mask=m)
      return off + jnp.sum(jnp.where(m, 1, 0))

    count = jnp.int32(0)
    for seg in range(SEG):
      pltpu.sync_copy(src_h.at[sid, seg, 0], sflat)
      pltpu.sync_copy(dst_h.at[sid, seg, 0], dflat)
      count = pl.loop(0, EPS // 16, init_carry=count)(compact)
    dummy = jnp.full((16,), TRASH, jnp.int32)
    for j in range(K // 16):
      packed[pl.ds(count + j * 16, 16)] = dummy
    nch = (count + (K - 1)) // K

    plsc.subcore_barrier()

    def unpack(c, b):
      base = c * K
      for j in range(K // 16):
        p = packed[pl.ds(base + j * 16, 16)]
        sidx[b, pl.ds(j * 16, 16)] = lax.shift_right_logical(p, SHIFT)
        didx[b, pl.ds(j * 16, 16)] = jnp.bitwise_and(p, MASK)

    def wait_gather(b):
      pltpu.make_async_copy(table.at[sidx.at[b]], rows.at[b], gsem).wait()

    def wait_scatter(b):
      pltpu.make_async_copy(rows.at[b], acc.at[didx.at[b]], ssem).wait()

    for b in range(LEAD):
      @pl.when(b < nch)
      def _():
        unpack(b, b)
        pltpu.async_copy(table.at[sidx.at[b]], rows.at[b], gsem)

    @pl.loop(0, CMAX, step=NBUF)
    def _main(c0):
      for b in range(NBUF):
        c = c0 + b

        @pl.when(c < nch)
        def _():
          wait_gather(b)
          pltpu.async_copy(rows.at[b], acc.at[didx.at[b]], ssem, add=True)
          if with_counts:
            pltpu.sync_copy(ones_v, cnt1.at[didx.at[b]], add=True)

        bf = (b + LEAD) % NBUF

        @pl.when(c + LEAD < nch)
        def _():
          @pl.when(c + LEAD >= NBUF)
          def _():
            wait_scatter(bf)
          unpack(c + LEAD, bf)
          pltpu.async_copy(table.at[sidx.at[bf]], rows.at[bf], gsem)

    for b in range(NBUF):
      cond = jnp.bool_(False)
      for t in range(1, NBUF + 1):
        cond = jnp.logical_or(
            cond, jnp.logical_and(nch >= t, (nch - t) % NBUF == b))

      @pl.when(cond)
      def _():
        wait_scatter(b)

    plsc.subcore_barrier()
    pltpu.sync_copy(acc.at[pl.ds(sid * ZR, ZR)],
                    out.at[cid, pl.ds(sid * ZR, ZR)])
    if with_counts:
      @pl.when(sid == 0)
      def _():
        pltpu.sync_copy(cnt1.at[pl.ds(0, HALF)], out_cnt.at[cid, 0])

  if with_counts:
    def agg(table, src_h, dst_h, zeros, zeros1, out, out_cnt, *scr):
      body((table, src_h, dst_h, zeros, zeros1, out, out_cnt) + scr)
  else:
    def agg(table, src_h, dst_h, zeros, out, *scr):
      body((table, src_h, dst_h, zeros, out) + scr)
    out_type = out_type[0]

  return pl.kernel(agg, out_type=out_type, mesh=mesh, scratch_types=scratch,
                   compiler_params=_SC_PARAMS)


_agg_cnt = _make_agg(True)
_agg = _make_agg(False)

R = 1000
GRID = N // R


def _dense1_body(acc, cnt, x, wl, wr, b, h_out, inv_out):
  inv = 1.0 / jnp.maximum(cnt[...], 1.0)
  agg = acc[...] * inv
  dn = (((1,), (1,)), ((), ()))
  y = (lax.dot_general(agg, wl[...], dn, preferred_element_type=jnp.float32)
       + lax.dot_general(x[...], wr[...], dn, preferred_element_type=jnp.float32)
       + b[...])
  h_out[...] = jnp.maximum(y, 0.0)
  inv_out[...] = jnp.broadcast_to(inv, (R, 8))


def _dense2_body(acc, h, inv8, wl, wr, b, out):
  agg = acc[...] * inv8[:, :1]
  dn = (((1,), (1,)), ((), ()))
  out[...] = (lax.dot_general(agg, wl[...], dn, preferred_element_type=jnp.float32)
              + lax.dot_general(h[...], wr[...], dn, preferred_element_type=jnp.float32)
              + b[...])


_dense1 = pl.pallas_call(
    _dense1_body,
    grid=(GRID,),
    in_specs=[
        pl.BlockSpec((R, D), lambda i: (i, 0)),
        pl.BlockSpec((R, 1), lambda i: (i, 0)),
        pl.BlockSpec((R, D), lambda i: (i, 0)),
        pl.BlockSpec((D, D), lambda i: (0, 0)),
        pl.BlockSpec((D, D), lambda i: (0, 0)),
        pl.BlockSpec((1, D), lambda i: (0, 0)),
    ],
    out_specs=[
        pl.BlockSpec((R, D), lambda i: (i, 0)),
        pl.BlockSpec((R, 8), lambda i: (i, 0)),
    ],
    out_shape=[
        jax.ShapeDtypeStruct((N, D), jnp.float32),
        jax.ShapeDtypeStruct((N, 8), jnp.float32),
    ],
)

_dense2 = pl.pallas_call(
    _dense2_body,
    grid=(GRID,),
    in_specs=[
        pl.BlockSpec((R, D), lambda i: (i, 0)),
        pl.BlockSpec((R, D), lambda i: (i, 0)),
        pl.BlockSpec((R, 8), lambda i: (i, 0)),
        pl.BlockSpec((D, D), lambda i: (0, 0)),
        pl.BlockSpec((D, D), lambda i: (0, 0)),
        pl.BlockSpec((1, D), lambda i: (0, 0)),
    ],
    out_specs=pl.BlockSpec((R, D), lambda i: (i, 0)),
    out_shape=jax.ShapeDtypeStruct((N, D), jnp.float32),
)


def kernel(x, edge_index, W1l, b1l, W1r, b1r, W2l, b2l, W2r, b2r):
  src_h = edge_index[0].astype(jnp.int32).reshape(NS, SEG, 1, EPS)
  dst_h = edge_index[1].astype(jnp.int32).reshape(NS, SEG, 1, EPS)
  zeros = jnp.zeros((ZR, D), jnp.float32)
  zeros1 = jnp.zeros((HALF,), jnp.float32)

  accp1, cntp = _agg_cnt(x, src_h, dst_h, zeros, zeros1)
  acc1 = accp1.reshape(NC * HALF, D)
  cnt = cntp.reshape(NC * HALF, 1)
  h, inv8 = _dense1(acc1, cnt, x, W1l, W1r, (b1l + b1r)[None, :])
  accp2 = _agg(h, src_h, dst_h, zeros)
  acc2 = accp2.reshape(NC * HALF, D)
  out = _dense2(acc2, h, inv8, W2l, W2r, (b2l + b2r)[None, :])
  return out
